# baseline (speedup 1.0000x reference)
import jax
import jax.numpy as jnp
from jax.experimental import pallas as pl
from jax.experimental.pallas import tpu as pltpu

B, N, F_IN, F_OUT, K = 4, 2048, 256, 256, 3
CHUNK = 256
NCH = N // CHUNK
NR = (B - 1) * N
XCH = NR // 4

_CD0 = (((0,), (0,)), ((), ()))


def _cheb_kernel(adj_hbm, data_hbm, w_ref, bias_ref, out_hbm,
                 adj_v, adj_bf, x0_v, z1_bf, dinv_v, diag_v,
                 xr_buf, outr_buf, out0_buf,
                 x0_sem, adj_sem, xr_sem, outw_sem):
    bias = bias_ref[:]
    w1_bf = w_ref[1].astype(jnp.bfloat16)
    w2x2_bf = (2.0 * w_ref[2]).astype(jnp.bfloat16)
    wc_bf = (w_ref[0] - w_ref[2]).astype(jnp.bfloat16)

    x0_copy = pltpu.make_async_copy(data_hbm.at[pl.ds(0, N), :], x0_v, x0_sem)
    x0_copy.start()
    adj_copies = []
    for i in range(NCH):
        c = pltpu.make_async_copy(adj_hbm.at[pl.ds(i * CHUNK, CHUNK), :],
                                  adj_v.at[pl.ds(i * CHUNK, CHUNK), :],
                                  adj_sem.at[i])
        c.start()
        adj_copies.append(c)
    xr_copies = []
    for i in range(4):
        c = pltpu.make_async_copy(data_hbm.at[pl.ds(N + i * XCH, XCH), :],
                                  xr_buf.at[i], xr_sem.at[i])
        c.start()
        xr_copies.append(c)

    x0_copy.wait()
    half = NCH // 2
    m1_parts = []
    for i in range(NCH):
        adj_copies[i].wait()
        sl = pl.ds(i * CHUNK, CHUNK)
        blk = adj_v[sl, :]
        rowsum = jnp.sum(blk, axis=1, keepdims=True)
        r = jax.lax.broadcasted_iota(jnp.int32, (CHUNK, N), 0) + i * CHUNK
        c = jax.lax.broadcasted_iota(jnp.int32, (CHUNK, N), 1)
        diag = jnp.sum(jnp.where(r == c, blk, 0.0), axis=1, keepdims=True)
        deg = rowsum - diag
        dinv = jnp.where(deg > 0, jax.lax.rsqrt(jnp.where(deg > 0, deg, 1.0)),
                         0.0)
        dinv_v[sl, :] = dinv
        diag_v[sl, :] = diag
        adj_bf[sl, :] = blk.astype(jnp.bfloat16)
        z1_bf[sl, :] = (dinv * x0_v[sl, :]).astype(jnp.bfloat16)
        if i % half == half - 1:
            hs = pl.ds((i - half + 1) * CHUNK, half * CHUNK)
            m1_parts.append(
                jax.lax.dot_general(adj_bf[hs, :], z1_bf[hs, :], _CD0,
                                    preferred_element_type=jnp.float32))

    dinv = dinv_v[:]
    diag = diag_v[:]
    m1 = m1_parts[0] + m1_parts[1]
    z1f = dinv * x0_v[:]
    t1 = dinv * (diag * z1f - m1)
    z2 = dinv * t1
    m2 = jax.lax.dot_general(adj_bf[:], z2.astype(jnp.bfloat16), _CD0,
                             preferred_element_type=jnp.float32)

    out_copies = []
    for i in range(4):
        xr_copies[i].wait()
        o = jnp.dot(xr_buf[i].astype(jnp.bfloat16), wc_bf,
                    preferred_element_type=jnp.float32) + bias
        outr_buf[i] = o
        c = pltpu.make_async_copy(outr_buf.at[i],
                                  out_hbm.at[pl.ds(N + i * XCH, XCH), :],
                                  outw_sem.at[i])
        c.start()
        out_copies.append(c)

    t2 = dinv * (diag * z2 - m2)
    out0_buf[:] = (
        jnp.dot(x0_v[:].astype(jnp.bfloat16), wc_bf,
                preferred_element_type=jnp.float32)
        + jnp.dot(t1.astype(jnp.bfloat16), w1_bf,
                  preferred_element_type=jnp.float32)
        + jnp.dot(t2.astype(jnp.bfloat16), w2x2_bf,
                  preferred_element_type=jnp.float32)
        + bias)
    c = pltpu.make_async_copy(out0_buf, out_hbm.at[pl.ds(0, N), :],
                              outw_sem.at[4])
    c.start()
    out_copies.append(c)
    for c in out_copies:
        c.wait()


def kernel(data, adj, W, bias):
    out = pl.pallas_call(
        _cheb_kernel,
        in_specs=[
            pl.BlockSpec(memory_space=pltpu.MemorySpace.HBM),
            pl.BlockSpec(memory_space=pltpu.MemorySpace.HBM),
            pl.BlockSpec(memory_space=pltpu.MemorySpace.VMEM),
            pl.BlockSpec(memory_space=pltpu.MemorySpace.VMEM),
        ],
        out_specs=pl.BlockSpec(memory_space=pltpu.MemorySpace.HBM),
        out_shape=jax.ShapeDtypeStruct((B * N, F_OUT), jnp.float32),
        scratch_shapes=[
            pltpu.VMEM((N, N), jnp.float32),
            pltpu.VMEM((N, N), jnp.bfloat16),
            pltpu.VMEM((N, F_IN), jnp.float32),
            pltpu.VMEM((N, F_IN), jnp.bfloat16),
            pltpu.VMEM((N, 1), jnp.float32),
            pltpu.VMEM((N, 1), jnp.float32),
            pltpu.VMEM((4, XCH, F_IN), jnp.float32),
            pltpu.VMEM((4, XCH, F_OUT), jnp.float32),
            pltpu.VMEM((N, F_OUT), jnp.float32),
            pltpu.SemaphoreType.DMA,
            pltpu.SemaphoreType.DMA((NCH,)),
            pltpu.SemaphoreType.DMA((4,)),
            pltpu.SemaphoreType.DMA((5,)),
        ],
    )(adj, data.reshape(B * N, F_IN), W, bias.reshape(1, F_OUT))
    return out.reshape(B, N, F_OUT)


# speedup vs baseline: 1.0180x; 1.0180x over previous
import jax
import jax.numpy as jnp
from jax.experimental import pallas as pl
from jax.experimental.pallas import tpu as pltpu

B, N, F_IN, F_OUT, K = 4, 2048, 256, 256, 3
CHUNK = 256
NCH = N // CHUNK
WAVE = NCH // 2
NR = (B - 1) * N
XCH = NR // 4

_CD0 = (((0,), (0,)), ((), ()))


def _cheb_kernel(adj_hbm, data_hbm, w_ref, bias_ref, out_hbm,
                 adj_v, adj_bf, x0_v, z1_bf, dinv_v, a1_v,
                 xr_buf, outr_buf, out0_buf,
                 x0_sem, adj_sem, xr_sem, outw_sem):
    bias = bias_ref[:]
    w1_bf = w_ref[1].astype(jnp.bfloat16)
    w2x2_bf = (2.0 * w_ref[2]).astype(jnp.bfloat16)
    wc_bf = (w_ref[0] - w_ref[2]).astype(jnp.bfloat16)
    ones_bf = jnp.ones((N, 128), jnp.bfloat16)

    x0_copy = pltpu.make_async_copy(data_hbm.at[pl.ds(0, N), :], x0_v, x0_sem)
    x0_copy.start()

    def adj_copy(i):
        return pltpu.make_async_copy(adj_hbm.at[pl.ds(i * CHUNK, CHUNK), :],
                                     adj_v.at[pl.ds(i * CHUNK, CHUNK), :],
                                     adj_sem.at[i])

    def xr_copy(i):
        return pltpu.make_async_copy(data_hbm.at[pl.ds(N + i * XCH, XCH), :],
                                     xr_buf.at[i], xr_sem.at[i])

    adj_copies = [adj_copy(i) for i in range(NCH)]
    xr_copies = [xr_copy(i) for i in range(4)]
    for i in range(WAVE):
        adj_copies[i].start()

    x0_copy.wait()
    m1_parts = []
    for i in range(NCH):
        adj_copies[i].wait()
        if i < WAVE:
            adj_copies[WAVE + i].start()
        if i < 4:
            xr_copies[i].start()
        sl = pl.ds(i * CHUNK, CHUNK)
        blk = adj_v[sl, :]
        blk_bf = blk.astype(jnp.bfloat16)
        adj_bf[sl, :] = blk_bf
        tile = blk[:, i * CHUNK:(i + 1) * CHUNK]
        r = jax.lax.broadcasted_iota(jnp.int32, (CHUNK, CHUNK), 0)
        c = jax.lax.broadcasted_iota(jnp.int32, (CHUNK, CHUNK), 1)
        diag = jnp.sum(jnp.where(r == c, tile, 0.0), axis=1, keepdims=True)
        rowsum = jnp.dot(blk_bf, ones_bf,
                         preferred_element_type=jnp.float32)[:, :1]
        deg = rowsum - diag
        dinv = jnp.where(deg > 0, jax.lax.rsqrt(jnp.where(deg > 0, deg, 1.0)),
                         0.0)
        dinv_v[sl, :] = dinv
        a1_v[sl, :] = dinv * dinv * diag
        z1_bf[sl, :] = (dinv * x0_v[sl, :]).astype(jnp.bfloat16)
        if i % WAVE == WAVE - 1:
            hs = pl.ds((i - WAVE + 1) * CHUNK, WAVE * CHUNK)
            m1_parts.append(
                jax.lax.dot_general(adj_bf[hs, :], z1_bf[hs, :], _CD0,
                                    preferred_element_type=jnp.float32))

    dinv = dinv_v[:]
    a1 = a1_v[:]
    m1 = m1_parts[0] + m1_parts[1]
    x0 = x0_v[:]
    t1 = a1 * x0 - dinv * m1
    z2 = dinv * t1
    m2 = jax.lax.dot_general(adj_bf[:], z2.astype(jnp.bfloat16), _CD0,
                             preferred_element_type=jnp.float32)

    out_copies = []
    for i in range(4):
        xr_copies[i].wait()
        o = jnp.dot(xr_buf[i].astype(jnp.bfloat16), wc_bf,
                    preferred_element_type=jnp.float32) + bias
        outr_buf[i] = o
        c = pltpu.make_async_copy(outr_buf.at[i],
                                  out_hbm.at[pl.ds(N + i * XCH, XCH), :],
                                  outw_sem.at[i])
        c.start()
        out_copies.append(c)

    t2 = a1 * t1 - dinv * m2
    out0_buf[:] = (
        jnp.dot(x0.astype(jnp.bfloat16), wc_bf,
                preferred_element_type=jnp.float32)
        + jnp.dot(t1.astype(jnp.bfloat16), w1_bf,
                  preferred_element_type=jnp.float32)
        + jnp.dot(t2.astype(jnp.bfloat16), w2x2_bf,
                  preferred_element_type=jnp.float32)
        + bias)
    c = pltpu.make_async_copy(out0_buf, out_hbm.at[pl.ds(0, N), :],
                              outw_sem.at[4])
    c.start()
    out_copies.append(c)
    for c in out_copies:
        c.wait()


def kernel(data, adj, W, bias):
    out = pl.pallas_call(
        _cheb_kernel,
        in_specs=[
            pl.BlockSpec(memory_space=pltpu.MemorySpace.HBM),
            pl.BlockSpec(memory_space=pltpu.MemorySpace.HBM),
            pl.BlockSpec(memory_space=pltpu.MemorySpace.VMEM),
            pl.BlockSpec(memory_space=pltpu.MemorySpace.VMEM),
        ],
        out_specs=pl.BlockSpec(memory_space=pltpu.MemorySpace.HBM),
        out_shape=jax.ShapeDtypeStruct((B * N, F_OUT), jnp.float32),
        scratch_shapes=[
            pltpu.VMEM((N, N), jnp.float32),
            pltpu.VMEM((N, N), jnp.bfloat16),
            pltpu.VMEM((N, F_IN), jnp.float32),
            pltpu.VMEM((N, F_IN), jnp.bfloat16),
            pltpu.VMEM((N, 1), jnp.float32),
            pltpu.VMEM((N, 1), jnp.float32),
            pltpu.VMEM((4, XCH, F_IN), jnp.float32),
            pltpu.VMEM((4, XCH, F_OUT), jnp.float32),
            pltpu.VMEM((N, F_OUT), jnp.float32),
            pltpu.SemaphoreType.DMA,
            pltpu.SemaphoreType.DMA((NCH,)),
            pltpu.SemaphoreType.DMA((4,)),
            pltpu.SemaphoreType.DMA((5,)),
        ],
    )(adj, data.reshape(B * N, F_IN), W, bias.reshape(1, F_OUT))
    return out.reshape(B, N, F_OUT)
